# baseline (speedup 1.0000x reference)
import functools
import math

import jax
import jax.numpy as jnp
import numpy as np
from jax import lax
from jax.experimental import pallas as pl
from jax.experimental.pallas import tpu as pltpu
from jax.experimental.pallas import tpu_sc as plsc

NN = 10000
KK = 16
IFZ = 128
AHZ = 12
AFZ = 16
QPZ = 4
VPZ = 8

FP = 160
NPAD = 10240
ROWS = KK * NPAD
NW = 32
CHUNK = 128
PER_W = ROWS // NW
NCH = PER_W // CHUNK

BN = 80
NBLK = NN // BN
BE = KK * BN

WC = math.sqrt(2.0 / (9.0 * QPZ))
WL = math.sqrt(1.0 / 3.0)


def _sc_gather(table, idx, rows):
    mesh = plsc.VectorSubcoreMesh(core_axis_name="c", subcore_axis_name="s")
    per_w = rows // NW
    nch = per_w // CHUNK

    @functools.partial(
        pl.kernel,
        mesh=mesh,
        compiler_params=pltpu.CompilerParams(use_tc_tiling_on_sc=False),
        out_type=jax.ShapeDtypeStruct((rows, FP), jnp.float32),
        scratch_types=[
            pltpu.VMEM((per_w,), jnp.int32),
            pltpu.VMEM((CHUNK, FP), jnp.float32),
            pltpu.VMEM((CHUNK, FP), jnp.float32),
            pltpu.SemaphoreType.DMA,
            pltpu.SemaphoreType.DMA,
            pltpu.SemaphoreType.DMA,
            pltpu.SemaphoreType.DMA,
        ],
    )
    def k(table_hbm, idx_hbm, out_hbm, idx_all, r0, r1, sg0, sg1, so0, so1):
        wid = lax.axis_index("s") * 2 + lax.axis_index("c")
        base = wid * per_w
        pltpu.sync_copy(idx_hbm.at[pl.ds(base, per_w)], idx_all)

        def ix(c):
            return idx_all.at[pl.ds(c * CHUNK, CHUNK)]

        def start_gather(c, r, sg):
            pltpu.async_copy(table_hbm.at[ix(c)], r, sg)

        def wait_gather(c, r, sg):
            pltpu.make_async_copy(table_hbm.at[ix(c)], r, sg).wait()

        def start_out(c, r, so):
            pltpu.async_copy(r, out_hbm.at[pl.ds(base + c * CHUNK, CHUNK)], so)

        def wait_out(c, r, so):
            pltpu.make_async_copy(
                r, out_hbm.at[pl.ds(base + c * CHUNK, CHUNK)], so).wait()

        npair = nch // 2
        start_gather(0, r0, sg0)

        def body(p, carry):
            c0 = 2 * p
            wait_gather(c0, r0, sg0)

            @pl.when(p > 0)
            def _():
                wait_out(c0 - 1, r1, so1)

            start_gather(c0 + 1, r1, sg1)
            start_out(c0, r0, so0)
            wait_gather(c0 + 1, r1, sg1)
            wait_out(c0, r0, so0)

            @pl.when(p < npair - 1)
            def _():
                start_gather(c0 + 2, r0, sg0)

            start_out(c0 + 1, r1, so1)
            return carry

        lax.fori_loop(0, npair, body, 0)
        wait_out(nch - 1, r1, so1)

    return k(table, idx)


def _dot(a, b):
    return jnp.dot(a, b, preferred_element_type=jnp.float32)


def _ksum(arr):
    acc = arr[0:BN]
    for k in range(1, KK):
        acc = acc + arr[k * BN:(k + 1) * BN]
    return acc


def _tc_body(g, x1d, x2km, affd, ped, wsrc, wdst, biaTs, bwp, bb,
             lng, lnb, b192s, b48s, t8, tile64, s12, ebc, o_ref):
    x1 = x1d[...]
    aff = affd[...]

    def A(i):
        return aff[:, i:i + 1]

    gall = g[...].reshape(BE, FP)
    xe = gall[:, 0:IFZ]
    aux = gall[:, 128:140]
    pe = gall[:, 140:148]

    proj = _dot(xe, wsrc[...])
    cT = _dot(jnp.cos(pe), t8[...])
    sT = _dot(jnp.sin(pe), t8[...])
    rk1 = proj[:, 0:192] * cT + proj[:, 816:1008] * sT

    qp = _dot(x1, wdst[...])
    cTd = _dot(jnp.cos(ped[...]), t8[...])
    sTd = _dot(jnp.sin(ped[...]), t8[...])
    rq1d = qp[:, 0:192] * cTd + qp[:, 192:384] * sTd
    rq1 = jnp.concatenate([rq1d] * KK, axis=0)
    qpt = jnp.concatenate([qp] * KK, axis=0)

    rb = _dot(aux, s12[...])

    def RB(j, w):
        return rb[:, 128 * j:128 * j + w]

    ux = qpt[:, 384:432] - proj[:, 384:432]
    uy = qpt[:, 432:480] - proj[:, 432:480]
    uz = qpt[:, 480:528] - proj[:, 480:528]

    g00 = A(0) * A(0) + A(4) * A(4) + A(8) * A(8)
    g11 = A(1) * A(1) + A(5) * A(5) + A(9) * A(9)
    g22 = A(2) * A(2) + A(6) * A(6) + A(10) * A(10)
    g01 = A(0) * A(1) + A(4) * A(5) + A(8) * A(9)
    g02 = A(0) * A(2) + A(4) * A(6) + A(8) * A(10)
    g12 = A(1) * A(2) + A(5) * A(6) + A(9) * A(10)
    gv = jnp.concatenate(
        [g00, g11, g22, 2.0 * g01, 2.0 * g02, 2.0 * g12], axis=1)
    gt = jnp.concatenate([gv] * KK, axis=0)

    s1 = _dot(rq1 * rk1, b192s[...])
    b48m = b48s[...]
    s2 = (gt[:, 0:1] * _dot(ux * ux, b48m)
          + gt[:, 1:2] * _dot(uy * uy, b48m)
          + gt[:, 2:3] * _dot(uz * uz, b48m)
          + gt[:, 3:4] * _dot(ux * uy, b48m)
          + gt[:, 4:5] * _dot(ux * uz, b48m)
          + gt[:, 5:6] * _dot(uy * uz, b48m))
    x2all = x2km[...].reshape(BE, 64)
    bias = _dot(x2all, biaTs[...])
    e = jnp.exp(s1 + s2 + bias)

    w_bc = _dot(e, ebc[...])
    p1 = w_bc[:, 0:192] * proj[:, 192:384]
    tvx = RB(0, 96) * proj[:, 528:624] + RB(1, 96) * proj[:, 624:720] \
        + RB(2, 96) * proj[:, 720:816] + RB(9, 96)
    tvy = RB(3, 96) * proj[:, 528:624] + RB(4, 96) * proj[:, 624:720] \
        + RB(5, 96) * proj[:, 720:816] + RB(10, 96)
    tvz = RB(6, 96) * proj[:, 528:624] + RB(7, 96) * proj[:, 624:720] \
        + RB(8, 96) * proj[:, 720:816] + RB(11, 96)
    pvx = w_bc[:, 1024:1120] * tvx
    pvy = w_bc[:, 1024:1120] * tvy
    pvz = w_bc[:, 1024:1120] * tvz
    p2 = w_bc[:, 256:1024] * _dot(x2all, tile64[...])

    denom = _ksum(e)
    acc1 = _ksum(p1)
    avx = _ksum(pvx)
    avy = _ksum(pvy)
    avz = _ksum(pvz)
    acc2 = _ksum(p2)

    winv = 1.0 / denom
    v_bc = _dot(winv, ebc[...])
    out1 = acc1 * v_bc[:, 0:192]
    out2 = acc2 * v_bc[:, 256:1024]
    w96 = v_bc[:, 1024:1120]
    yx = avx * w96 - A(3)
    yy = avy * w96 - A(7)
    yz = avz * w96 - A(11)
    ox = A(0) * yx + A(4) * yy + A(8) * yz
    oy = A(1) * yx + A(5) * yy + A(9) * yz
    oz = A(2) * yx + A(6) * yy + A(10) * yz
    nrm = jnp.sqrt(ox * ox + oy * oy + oz * oz + 1e-12)

    cat = jnp.concatenate([out1, out2, ox, oy, oz, nrm], axis=1)
    out = _dot(cat, bwp[...]) + bb[...]
    h = math.sqrt(2.0) * x1 + out
    mu = jnp.mean(h, axis=-1, keepdims=True)
    var = jnp.mean(jnp.square(h - mu), axis=-1, keepdims=True)
    o_ref[...] = lng[...] * (h - mu) * jax.lax.rsqrt(var + 1e-5) + lnb[...]


def _tc_specs(nblk):
    def blk(shape, imap):
        return pl.BlockSpec(shape, imap)

    def c2(shape):
        return pl.BlockSpec(shape, lambda i: (0, 0))

    in_specs = [
        blk((KK, BN, FP), lambda i: (0, i, 0)),
        blk((BN, IFZ), lambda i: (i, 0)),
        blk((KK, BN, 64), lambda i: (0, i, 0)),
        blk((BN, 16), lambda i: (i, 0)),
        blk((BN, 8), lambda i: (i, 0)),
        c2((IFZ, 1008)),
        c2((IFZ, 528)),
        c2((64, AHZ)),
        c2((1344, IFZ)),
        c2((1, IFZ)),
        c2((1, IFZ)),
        c2((1, IFZ)),
        c2((192, AHZ)),
        c2((48, AHZ)),
        c2((8, 192)),
        c2((64, 768)),
        c2((12, 1536)),
        c2((AHZ, 1120)),
    ]
    out_spec = pl.BlockSpec((BN, IFZ), lambda i: (i, 0))
    out_shape = jax.ShapeDtypeStruct((nblk * BN, IFZ), jnp.float32)
    return (nblk,), in_specs, out_spec, out_shape


def _prep_consts():
    b192 = (np.arange(192)[:, None] // AFZ == np.arange(AHZ)[None, :]
            ).astype(np.float32)
    b48 = (np.arange(48)[:, None] // QPZ == np.arange(AHZ)[None, :]
           ).astype(np.float32)

    ebc = np.zeros((AHZ, 1120), np.float32)
    for a in range(AHZ):
        ebc[a, a * 16:(a + 1) * 16] = 1.0
        ebc[a, 256 + a * 64:256 + (a + 1) * 64] = 1.0
        ebc[a, 1024 + a * 8:1024 + (a + 1) * 8] = 1.0

    t8 = np.zeros((8, 192), np.float32)
    for a in range(AHZ):
        for u in range(16):
            t8[u % 8, a * 16 + u] = 1.0

    tile64 = np.zeros((64, 768), np.float32)
    for a in range(AHZ):
        for i in range(64):
            tile64[i, a * 64 + i] = 1.0

    s12 = np.zeros((12, 1536), np.float32)
    for j in range(12):
        s12[j, 128 * j:128 * j + 96] = 1.0

    pswap = np.zeros((192, 192), np.float32)
    for a in range(AHZ):
        for t in range(8):
            pswap[a * 16 + 8 + t, a * 16 + t] = -1.0
            pswap[a * 16 + t, a * 16 + 8 + t] = 1.0

    rope_perm = np.zeros(192, np.int32)
    for a in range(AHZ):
        for t in range(8):
            rope_perm[a * 16 + t] = a * 16 + 2 * t
            rope_perm[a * 16 + 8 + t] = a * 16 + 2 * t + 1

    qk_perm = np.zeros(144, np.int32)
    for d in range(3):
        for a in range(AHZ):
            for p in range(QPZ):
                qk_perm[d * 48 + a * QPZ + p] = a * QPZ * 3 + p * 3 + d
    v_perm = np.zeros(288, np.int32)
    for d in range(3):
        for a in range(AHZ):
            for v in range(VPZ):
                v_perm[d * 96 + a * VPZ + v] = a * VPZ * 3 + v * 3 + d

    colperm = np.arange(1344)
    for d in range(3):
        for a in range(AHZ):
            for v in range(VPZ):
                colperm[960 + d * 96 + a * VPZ + v] = 960 + a * VPZ * 3 + v * 3 + d

    return (b192, b48, ebc, t8, tile64, s12, pswap, rope_perm, qk_perm,
            v_perm, colperm)


_CONSTS = _prep_consts()


def _weights(q1_w, q2_w, k1_w, k2_w, v1_w, v2_w, bia_w, back_w, back_b,
             gamma, ln_g, ln_b):
    (b192, b48, ebc, t8, tile64, s12, pswap, rope_perm, qk_perm,
     v_perm, colperm) = _CONSTS
    psw = jnp.asarray(pswap)

    wk1 = k1_w[rope_perm].T
    wsrc = jnp.concatenate(
        [wk1, v1_w.T, k2_w[qk_perm].T, v2_w[v_perm].T, _dot(wk1, psw)],
        axis=1)
    wq1 = q1_w[rope_perm].T
    wdst = jnp.concatenate([wq1, _dot(wq1, psw), q2_w[qk_perm].T], axis=1)
    bwp = back_w[:, colperm].T

    b192s = jnp.asarray(b192) * (WL * 0.25)
    b48s = jnp.asarray(b48) * (gamma.reshape(1, AHZ) * (-0.1 * WC * WL))
    biaTs = bia_w.T * WL

    return (wsrc, wdst, biaTs, bwp, back_b.reshape(1, IFZ),
            ln_g.reshape(1, IFZ), ln_b.reshape(1, IFZ),
            b192s, b48s, jnp.asarray(t8), jnp.asarray(tile64),
            jnp.asarray(s12), jnp.asarray(ebc))


def _tc_call(g, x1h, x2kmh, affdh, pedh, wts, nblk):
    grid, in_specs, out_spec, out_shape = _tc_specs(nblk)
    return pl.pallas_call(
        _tc_body, grid=grid, in_specs=in_specs, out_specs=out_spec,
        out_shape=out_shape,
    )(g, x1h, x2kmh, affdh, pedh, *wts)


NH = (5040, 4960)
NPAD_H = 5120


def kernel(x1, x2, affines, pos_emb, edge_index, q1_w, k1_w, v1_w, q2_w,
           k2_w, v2_w, bia_w, back_w, back_b, gamma, ln_g, ln_b):
    rot = affines[:, :3, :3].reshape(NN, 9)
    trans = affines[:, :3, 3]
    table = jnp.concatenate(
        [x1, rot, trans, pos_emb,
         jnp.zeros((NN, FP - IFZ - 20), jnp.float32)], axis=1)

    idxT = edge_index.T.astype(jnp.int32)
    x2km = x2.transpose(1, 0, 2)
    affd = affines.reshape(NN, 16)
    wts = _weights(q1_w, q2_w, k1_w, k2_w, v1_w, v2_w, bia_w, back_w,
                   back_b, gamma, ln_g, ln_b)

    ys = []
    off = 0
    for nh in NH:
        sl = slice(off, off + nh)
        idx_h = jnp.pad(idxT[:, sl],
                        ((0, 0), (0, NPAD_H - nh))).reshape(KK * NPAD_H)
        g = _sc_gather(table, idx_h, KK * NPAD_H).reshape(KK, NPAD_H, FP)
        ys.append(_tc_call(g, x1[sl], x2km[:, sl], affd[sl], pos_emb[sl],
                           wts, nh // BN))
        off += nh
    return jnp.concatenate(ys, axis=0)


# speedup vs baseline: 1.0310x; 1.0310x over previous
import functools
import math

import jax
import jax.numpy as jnp
import numpy as np
from jax import lax
from jax.experimental import pallas as pl
from jax.experimental.pallas import tpu as pltpu
from jax.experimental.pallas import tpu_sc as plsc

NN = 10000
KK = 16
IFZ = 128
AHZ = 12
AFZ = 16
QPZ = 4
VPZ = 8

FP = 160
NPAD = 10240
ROWS = KK * NPAD
NW = 32
CHUNK = 128
PER_W = ROWS // NW
NCH = PER_W // CHUNK

BN = 80
NBLK = NN // BN
BE = KK * BN

WC = math.sqrt(2.0 / (9.0 * QPZ))
WL = math.sqrt(1.0 / 3.0)


def _sc_gather(table, idx, rows):
    mesh = plsc.VectorSubcoreMesh(core_axis_name="c", subcore_axis_name="s")
    per_w = rows // NW
    nch = per_w // CHUNK

    @functools.partial(
        pl.kernel,
        mesh=mesh,
        compiler_params=pltpu.CompilerParams(use_tc_tiling_on_sc=False),
        out_type=jax.ShapeDtypeStruct((rows, FP), jnp.float32),
        scratch_types=[
            pltpu.VMEM((per_w,), jnp.int32),
            pltpu.VMEM((CHUNK, FP), jnp.float32),
            pltpu.VMEM((CHUNK, FP), jnp.float32),
            pltpu.SemaphoreType.DMA,
            pltpu.SemaphoreType.DMA,
            pltpu.SemaphoreType.DMA,
            pltpu.SemaphoreType.DMA,
        ],
    )
    def k(table_hbm, idx_hbm, out_hbm, idx_all, r0, r1, sg0, sg1, so0, so1):
        wid = lax.axis_index("s") * 2 + lax.axis_index("c")
        base = wid * per_w
        pltpu.sync_copy(idx_hbm.at[pl.ds(base, per_w)], idx_all)

        def ix(c):
            return idx_all.at[pl.ds(c * CHUNK, CHUNK)]

        def start_gather(c, r, sg):
            pltpu.async_copy(table_hbm.at[ix(c)], r, sg)

        def wait_gather(c, r, sg):
            pltpu.make_async_copy(table_hbm.at[ix(c)], r, sg).wait()

        def start_out(c, r, so):
            pltpu.async_copy(r, out_hbm.at[pl.ds(base + c * CHUNK, CHUNK)], so)

        def wait_out(c, r, so):
            pltpu.make_async_copy(
                r, out_hbm.at[pl.ds(base + c * CHUNK, CHUNK)], so).wait()

        npair = nch // 2
        start_gather(0, r0, sg0)

        def body(p, carry):
            c0 = 2 * p
            wait_gather(c0, r0, sg0)

            @pl.when(p > 0)
            def _():
                wait_out(c0 - 1, r1, so1)

            start_gather(c0 + 1, r1, sg1)
            start_out(c0, r0, so0)
            wait_gather(c0 + 1, r1, sg1)
            wait_out(c0, r0, so0)

            @pl.when(p < npair - 1)
            def _():
                start_gather(c0 + 2, r0, sg0)

            start_out(c0 + 1, r1, so1)
            return carry

        lax.fori_loop(0, npair, body, 0)
        wait_out(nch - 1, r1, so1)

    return k(table, idx)


def _dot(a, b):
    return jnp.dot(a, b, preferred_element_type=jnp.float32)


def _ksum(arr):
    acc = arr[0:BN]
    for k in range(1, KK):
        acc = acc + arr[k * BN:(k + 1) * BN]
    return acc


def _tc_body(g, x1d, x2f, affd, ped, wsrc, wdst, biaTs, bwp, bb,
             lng, lnb, b192s, b48s, t8, tile64, s12, ebc, o_ref):
    x1 = x1d[...]
    aff = affd[...]

    def A(i):
        return aff[:, i:i + 1]

    gall = g[...].reshape(BE, FP)
    xe = gall[:, 0:IFZ]
    aux = gall[:, 128:140]
    pe = gall[:, 140:148]

    proj = _dot(xe, wsrc[...])
    cT = _dot(jnp.cos(pe), t8[...])
    sT = _dot(jnp.sin(pe), t8[...])
    rk1 = proj[:, 0:192] * cT + proj[:, 816:1008] * sT

    qp = _dot(x1, wdst[...])
    cTd = _dot(jnp.cos(ped[...]), t8[...])
    sTd = _dot(jnp.sin(ped[...]), t8[...])
    rq1d = qp[:, 0:192] * cTd + qp[:, 192:384] * sTd
    rq1 = jnp.concatenate([rq1d] * KK, axis=0)
    qpt = jnp.concatenate([qp] * KK, axis=0)

    rb = _dot(aux, s12[...])

    def RB(j, w):
        return rb[:, 128 * j:128 * j + w]

    ux = qpt[:, 384:432] - proj[:, 384:432]
    uy = qpt[:, 432:480] - proj[:, 432:480]
    uz = qpt[:, 480:528] - proj[:, 480:528]

    g00 = A(0) * A(0) + A(4) * A(4) + A(8) * A(8)
    g11 = A(1) * A(1) + A(5) * A(5) + A(9) * A(9)
    g22 = A(2) * A(2) + A(6) * A(6) + A(10) * A(10)
    g01 = A(0) * A(1) + A(4) * A(5) + A(8) * A(9)
    g02 = A(0) * A(2) + A(4) * A(6) + A(8) * A(10)
    g12 = A(1) * A(2) + A(5) * A(6) + A(9) * A(10)
    gv = jnp.concatenate(
        [g00, g11, g22, 2.0 * g01, 2.0 * g02, 2.0 * g12], axis=1)
    gt = jnp.concatenate([gv] * KK, axis=0)

    x2r = x2f[...]
    x2all = jnp.concatenate(
        [x2r[:, 64 * k:64 * (k + 1)] for k in range(KK)], axis=0)

    s1 = _dot(rq1 * rk1, b192s[...])
    b48m = b48s[...]
    s2 = (gt[:, 0:1] * _dot(ux * ux, b48m)
          + gt[:, 1:2] * _dot(uy * uy, b48m)
          + gt[:, 2:3] * _dot(uz * uz, b48m)
          + gt[:, 3:4] * _dot(ux * uy, b48m)
          + gt[:, 4:5] * _dot(ux * uz, b48m)
          + gt[:, 5:6] * _dot(uy * uz, b48m))
    bias = _dot(x2all, biaTs[...])
    e = jnp.exp(s1 + s2 + bias)

    w_bc = _dot(e, ebc[...])
    p1 = w_bc[:, 0:192] * proj[:, 192:384]
    tvx = RB(0, 96) * proj[:, 528:624] + RB(1, 96) * proj[:, 624:720] \
        + RB(2, 96) * proj[:, 720:816] + RB(9, 96)
    tvy = RB(3, 96) * proj[:, 528:624] + RB(4, 96) * proj[:, 624:720] \
        + RB(5, 96) * proj[:, 720:816] + RB(10, 96)
    tvz = RB(6, 96) * proj[:, 528:624] + RB(7, 96) * proj[:, 624:720] \
        + RB(8, 96) * proj[:, 720:816] + RB(11, 96)
    pvx = w_bc[:, 1024:1120] * tvx
    pvy = w_bc[:, 1024:1120] * tvy
    pvz = w_bc[:, 1024:1120] * tvz
    p2 = w_bc[:, 256:1024] * _dot(x2all, tile64[...])

    denom = _ksum(e)
    acc1 = _ksum(p1)
    avx = _ksum(pvx)
    avy = _ksum(pvy)
    avz = _ksum(pvz)
    acc2 = _ksum(p2)

    winv = 1.0 / denom
    v_bc = _dot(winv, ebc[...])
    out1 = acc1 * v_bc[:, 0:192]
    out2 = acc2 * v_bc[:, 256:1024]
    w96 = v_bc[:, 1024:1120]
    yx = avx * w96 - A(3)
    yy = avy * w96 - A(7)
    yz = avz * w96 - A(11)
    ox = A(0) * yx + A(4) * yy + A(8) * yz
    oy = A(1) * yx + A(5) * yy + A(9) * yz
    oz = A(2) * yx + A(6) * yy + A(10) * yz
    nrm = jnp.sqrt(ox * ox + oy * oy + oz * oz + 1e-12)

    cat = jnp.concatenate([out1, out2, ox, oy, oz, nrm], axis=1)
    out = _dot(cat, bwp[...]) + bb[...]
    h = math.sqrt(2.0) * x1 + out
    mu = jnp.mean(h, axis=-1, keepdims=True)
    var = jnp.mean(jnp.square(h - mu), axis=-1, keepdims=True)
    o_ref[...] = lng[...] * (h - mu) * jax.lax.rsqrt(var + 1e-5) + lnb[...]


def _tc_specs(nblk):
    def blk(shape, imap):
        return pl.BlockSpec(shape, imap)

    def c2(shape):
        return pl.BlockSpec(shape, lambda i: (0, 0))

    in_specs = [
        blk((KK, BN, FP), lambda i: (0, i, 0)),
        blk((BN, IFZ), lambda i: (i, 0)),
        blk((BN, KK * 64), lambda i: (i, 0)),
        blk((BN, 16), lambda i: (i, 0)),
        blk((BN, 8), lambda i: (i, 0)),
        c2((IFZ, 1008)),
        c2((IFZ, 528)),
        c2((64, AHZ)),
        c2((1344, IFZ)),
        c2((1, IFZ)),
        c2((1, IFZ)),
        c2((1, IFZ)),
        c2((192, AHZ)),
        c2((48, AHZ)),
        c2((8, 192)),
        c2((64, 768)),
        c2((12, 1536)),
        c2((AHZ, 1120)),
    ]
    out_spec = pl.BlockSpec((BN, IFZ), lambda i: (i, 0))
    out_shape = jax.ShapeDtypeStruct((nblk * BN, IFZ), jnp.float32)
    return (nblk,), in_specs, out_spec, out_shape


def _prep_consts():
    b192 = (np.arange(192)[:, None] // AFZ == np.arange(AHZ)[None, :]
            ).astype(np.float32)
    b48 = (np.arange(48)[:, None] // QPZ == np.arange(AHZ)[None, :]
           ).astype(np.float32)

    ebc = np.zeros((AHZ, 1120), np.float32)
    for a in range(AHZ):
        ebc[a, a * 16:(a + 1) * 16] = 1.0
        ebc[a, 256 + a * 64:256 + (a + 1) * 64] = 1.0
        ebc[a, 1024 + a * 8:1024 + (a + 1) * 8] = 1.0

    t8 = np.zeros((8, 192), np.float32)
    for a in range(AHZ):
        for u in range(16):
            t8[u % 8, a * 16 + u] = 1.0

    tile64 = np.zeros((64, 768), np.float32)
    for a in range(AHZ):
        for i in range(64):
            tile64[i, a * 64 + i] = 1.0

    s12 = np.zeros((12, 1536), np.float32)
    for j in range(12):
        s12[j, 128 * j:128 * j + 96] = 1.0

    pswap = np.zeros((192, 192), np.float32)
    for a in range(AHZ):
        for t in range(8):
            pswap[a * 16 + 8 + t, a * 16 + t] = -1.0
            pswap[a * 16 + t, a * 16 + 8 + t] = 1.0

    rope_perm = np.zeros(192, np.int32)
    for a in range(AHZ):
        for t in range(8):
            rope_perm[a * 16 + t] = a * 16 + 2 * t
            rope_perm[a * 16 + 8 + t] = a * 16 + 2 * t + 1

    qk_perm = np.zeros(144, np.int32)
    for d in range(3):
        for a in range(AHZ):
            for p in range(QPZ):
                qk_perm[d * 48 + a * QPZ + p] = a * QPZ * 3 + p * 3 + d
    v_perm = np.zeros(288, np.int32)
    for d in range(3):
        for a in range(AHZ):
            for v in range(VPZ):
                v_perm[d * 96 + a * VPZ + v] = a * VPZ * 3 + v * 3 + d

    colperm = np.arange(1344)
    for d in range(3):
        for a in range(AHZ):
            for v in range(VPZ):
                colperm[960 + d * 96 + a * VPZ + v] = 960 + a * VPZ * 3 + v * 3 + d

    return (b192, b48, ebc, t8, tile64, s12, pswap, rope_perm, qk_perm,
            v_perm, colperm)


_CONSTS = _prep_consts()


def _weights(q1_w, q2_w, k1_w, k2_w, v1_w, v2_w, bia_w, back_w, back_b,
             gamma, ln_g, ln_b):
    (b192, b48, ebc, t8, tile64, s12, pswap, rope_perm, qk_perm,
     v_perm, colperm) = _CONSTS
    psw = jnp.asarray(pswap)

    wk1 = k1_w[rope_perm].T
    wsrc = jnp.concatenate(
        [wk1, v1_w.T, k2_w[qk_perm].T, v2_w[v_perm].T, _dot(wk1, psw)],
        axis=1)
    wq1 = q1_w[rope_perm].T
    wdst = jnp.concatenate([wq1, _dot(wq1, psw), q2_w[qk_perm].T], axis=1)
    bwp = back_w[:, colperm].T

    b192s = jnp.asarray(b192) * (WL * 0.25)
    b48s = jnp.asarray(b48) * (gamma.reshape(1, AHZ) * (-0.1 * WC * WL))
    biaTs = bia_w.T * WL

    return (wsrc, wdst, biaTs, bwp, back_b.reshape(1, IFZ),
            ln_g.reshape(1, IFZ), ln_b.reshape(1, IFZ),
            b192s, b48s, jnp.asarray(t8), jnp.asarray(tile64),
            jnp.asarray(s12), jnp.asarray(ebc))


def _tc_call(g, x1h, x2fh, affdh, pedh, wts, nblk):
    grid, in_specs, out_spec, out_shape = _tc_specs(nblk)
    return pl.pallas_call(
        _tc_body, grid=grid, in_specs=in_specs, out_specs=out_spec,
        out_shape=out_shape,
    )(g, x1h, x2fh, affdh, pedh, *wts)


def kernel(x1, x2, affines, pos_emb, edge_index, q1_w, k1_w, v1_w, q2_w,
           k2_w, v2_w, bia_w, back_w, back_b, gamma, ln_g, ln_b):
    rot = affines[:, :3, :3].reshape(NN, 9)
    trans = affines[:, :3, 3]
    table = jnp.concatenate(
        [x1, rot, trans, pos_emb,
         jnp.zeros((NN, FP - IFZ - 20), jnp.float32)], axis=1)

    idxT = edge_index.T.astype(jnp.int32)
    idx = jnp.pad(idxT, ((0, 0), (0, NPAD - NN))).reshape(ROWS)
    wts = _weights(q1_w, q2_w, k1_w, k2_w, v1_w, v2_w, bia_w, back_w,
                   back_b, gamma, ln_g, ln_b)

    g = _sc_gather(table, idx, ROWS).reshape(KK, NPAD, FP)
    return _tc_call(g, x1, x2.reshape(NN, KK * 64), affines.reshape(NN, 16),
                    pos_emb, wts, NBLK)


# speedup vs baseline: 1.0433x; 1.0120x over previous
import functools
import math

import jax
import jax.numpy as jnp
import numpy as np
from jax import lax
from jax.experimental import pallas as pl
from jax.experimental.pallas import tpu as pltpu
from jax.experimental.pallas import tpu_sc as plsc

NN = 10000
KK = 16
IFZ = 128
AHZ = 12
AFZ = 16
QPZ = 4
VPZ = 8

FP = 160
NPAD = 10240
ROWS = KK * NPAD
NW = 32
CHUNK = 128
PER_W = ROWS // NW
NCH = PER_W // CHUNK

BN = 200
NBLK = NN // BN
BE = KK * BN

WC = math.sqrt(2.0 / (9.0 * QPZ))
WL = math.sqrt(1.0 / 3.0)


def _sc_gather(table, idx, rows):
    mesh = plsc.VectorSubcoreMesh(core_axis_name="c", subcore_axis_name="s")
    per_w = rows // NW
    nch = per_w // CHUNK

    @functools.partial(
        pl.kernel,
        mesh=mesh,
        compiler_params=pltpu.CompilerParams(use_tc_tiling_on_sc=False),
        out_type=jax.ShapeDtypeStruct((rows, FP), jnp.float32),
        scratch_types=[
            pltpu.VMEM((per_w,), jnp.int32),
            pltpu.VMEM((CHUNK, FP), jnp.float32),
            pltpu.VMEM((CHUNK, FP), jnp.float32),
            pltpu.SemaphoreType.DMA,
            pltpu.SemaphoreType.DMA,
            pltpu.SemaphoreType.DMA,
            pltpu.SemaphoreType.DMA,
        ],
    )
    def k(table_hbm, idx_hbm, out_hbm, idx_all, r0, r1, sg0, sg1, so0, so1):
        wid = lax.axis_index("s") * 2 + lax.axis_index("c")
        base = wid * per_w
        pltpu.sync_copy(idx_hbm.at[pl.ds(base, per_w)], idx_all)

        def ix(c):
            return idx_all.at[pl.ds(c * CHUNK, CHUNK)]

        def start_gather(c, r, sg):
            pltpu.async_copy(table_hbm.at[ix(c)], r, sg)

        def wait_gather(c, r, sg):
            pltpu.make_async_copy(table_hbm.at[ix(c)], r, sg).wait()

        def start_out(c, r, so):
            pltpu.async_copy(r, out_hbm.at[pl.ds(base + c * CHUNK, CHUNK)], so)

        def wait_out(c, r, so):
            pltpu.make_async_copy(
                r, out_hbm.at[pl.ds(base + c * CHUNK, CHUNK)], so).wait()

        npair = nch // 2
        start_gather(0, r0, sg0)

        def body(p, carry):
            c0 = 2 * p
            wait_gather(c0, r0, sg0)

            @pl.when(p > 0)
            def _():
                wait_out(c0 - 1, r1, so1)

            start_gather(c0 + 1, r1, sg1)
            start_out(c0, r0, so0)
            wait_gather(c0 + 1, r1, sg1)
            wait_out(c0, r0, so0)

            @pl.when(p < npair - 1)
            def _():
                start_gather(c0 + 2, r0, sg0)

            start_out(c0 + 1, r1, so1)
            return carry

        lax.fori_loop(0, npair, body, 0)
        wait_out(nch - 1, r1, so1)

    return k(table, idx)


def _dot(a, b):
    return jnp.dot(a, b, preferred_element_type=jnp.float32)


def _ksum(arr):
    acc = arr[0:BN]
    for k in range(1, KK):
        acc = acc + arr[k * BN:(k + 1) * BN]
    return acc


def _tc_body(g, x1d, x2f, affd, ped, wsrc, wdst, biaTs, bwp, bb,
             lng, lnb, b192s, b48s, t8, tile64, s12, ebc, o_ref):
    x1 = x1d[...]
    aff = affd[...]

    def A(i):
        return aff[:, i:i + 1]

    gall = g[...].reshape(BE, FP)
    xe = gall[:, 0:IFZ]
    aux = gall[:, 128:140]
    pe = gall[:, 140:148]

    proj = _dot(xe, wsrc[...])
    cT = _dot(jnp.cos(pe), t8[...])
    sT = _dot(jnp.sin(pe), t8[...])
    rk1 = proj[:, 0:192] * cT + proj[:, 816:1008] * sT

    qp = _dot(x1, wdst[...])
    cTd = _dot(jnp.cos(ped[...]), t8[...])
    sTd = _dot(jnp.sin(ped[...]), t8[...])
    rq1d = qp[:, 0:192] * cTd + qp[:, 192:384] * sTd
    rq1 = jnp.concatenate([rq1d] * KK, axis=0)
    qpt = jnp.concatenate([qp] * KK, axis=0)

    rb = _dot(aux, s12[...])

    def RB(j, w):
        return rb[:, 128 * j:128 * j + w]

    ux = qpt[:, 384:432] - proj[:, 384:432]
    uy = qpt[:, 432:480] - proj[:, 432:480]
    uz = qpt[:, 480:528] - proj[:, 480:528]

    g00 = A(0) * A(0) + A(4) * A(4) + A(8) * A(8)
    g11 = A(1) * A(1) + A(5) * A(5) + A(9) * A(9)
    g22 = A(2) * A(2) + A(6) * A(6) + A(10) * A(10)
    g01 = A(0) * A(1) + A(4) * A(5) + A(8) * A(9)
    g02 = A(0) * A(2) + A(4) * A(6) + A(8) * A(10)
    g12 = A(1) * A(2) + A(5) * A(6) + A(9) * A(10)
    gv = jnp.concatenate(
        [g00, g11, g22, 2.0 * g01, 2.0 * g02, 2.0 * g12], axis=1)
    gt = jnp.concatenate([gv] * KK, axis=0)

    x2r = x2f[...]
    x2all = jnp.concatenate(
        [x2r[:, 64 * k:64 * (k + 1)] for k in range(KK)], axis=0)

    s1 = _dot(rq1 * rk1, b192s[...])
    b48m = b48s[...]
    s2 = (gt[:, 0:1] * _dot(ux * ux, b48m)
          + gt[:, 1:2] * _dot(uy * uy, b48m)
          + gt[:, 2:3] * _dot(uz * uz, b48m)
          + gt[:, 3:4] * _dot(ux * uy, b48m)
          + gt[:, 4:5] * _dot(ux * uz, b48m)
          + gt[:, 5:6] * _dot(uy * uz, b48m))
    bias = _dot(x2all, biaTs[...])
    e = jnp.exp(s1 + s2 + bias)

    w_bc = _dot(e, ebc[...])
    p1 = w_bc[:, 0:192] * proj[:, 192:384]
    tvx = RB(0, 96) * proj[:, 528:624] + RB(1, 96) * proj[:, 624:720] \
        + RB(2, 96) * proj[:, 720:816] + RB(9, 96)
    tvy = RB(3, 96) * proj[:, 528:624] + RB(4, 96) * proj[:, 624:720] \
        + RB(5, 96) * proj[:, 720:816] + RB(10, 96)
    tvz = RB(6, 96) * proj[:, 528:624] + RB(7, 96) * proj[:, 624:720] \
        + RB(8, 96) * proj[:, 720:816] + RB(11, 96)
    pvx = w_bc[:, 1024:1120] * tvx
    pvy = w_bc[:, 1024:1120] * tvy
    pvz = w_bc[:, 1024:1120] * tvz
    p2 = w_bc[:, 256:1024] * _dot(x2all, tile64[...])

    denom = _ksum(e)
    acc1 = _ksum(p1)
    avx = _ksum(pvx)
    avy = _ksum(pvy)
    avz = _ksum(pvz)
    acc2 = _ksum(p2)

    winv = 1.0 / denom
    v_bc = _dot(winv, ebc[...])
    out1 = acc1 * v_bc[:, 0:192]
    out2 = acc2 * v_bc[:, 256:1024]
    w96 = v_bc[:, 1024:1120]
    yx = avx * w96 - A(3)
    yy = avy * w96 - A(7)
    yz = avz * w96 - A(11)
    ox = A(0) * yx + A(4) * yy + A(8) * yz
    oy = A(1) * yx + A(5) * yy + A(9) * yz
    oz = A(2) * yx + A(6) * yy + A(10) * yz
    nrm = jnp.sqrt(ox * ox + oy * oy + oz * oz + 1e-12)

    cat = jnp.concatenate([out1, out2, ox, oy, oz, nrm], axis=1)
    out = _dot(cat, bwp[...]) + bb[...]
    h = math.sqrt(2.0) * x1 + out
    mu = jnp.mean(h, axis=-1, keepdims=True)
    var = jnp.mean(jnp.square(h - mu), axis=-1, keepdims=True)
    o_ref[...] = lng[...] * (h - mu) * jax.lax.rsqrt(var + 1e-5) + lnb[...]


def _tc_specs(nblk):
    def blk(shape, imap):
        return pl.BlockSpec(shape, imap)

    def c2(shape):
        return pl.BlockSpec(shape, lambda i: (0, 0))

    in_specs = [
        blk((KK, BN, FP), lambda i: (0, i, 0)),
        blk((BN, IFZ), lambda i: (i, 0)),
        blk((BN, KK * 64), lambda i: (i, 0)),
        blk((BN, 16), lambda i: (i, 0)),
        blk((BN, 8), lambda i: (i, 0)),
        c2((IFZ, 1008)),
        c2((IFZ, 528)),
        c2((64, AHZ)),
        c2((1344, IFZ)),
        c2((1, IFZ)),
        c2((1, IFZ)),
        c2((1, IFZ)),
        c2((192, AHZ)),
        c2((48, AHZ)),
        c2((8, 192)),
        c2((64, 768)),
        c2((12, 1536)),
        c2((AHZ, 1120)),
    ]
    out_spec = pl.BlockSpec((BN, IFZ), lambda i: (i, 0))
    out_shape = jax.ShapeDtypeStruct((nblk * BN, IFZ), jnp.float32)
    return (nblk,), in_specs, out_spec, out_shape


def _prep_consts():
    b192 = (np.arange(192)[:, None] // AFZ == np.arange(AHZ)[None, :]
            ).astype(np.float32)
    b48 = (np.arange(48)[:, None] // QPZ == np.arange(AHZ)[None, :]
           ).astype(np.float32)

    ebc = np.zeros((AHZ, 1120), np.float32)
    for a in range(AHZ):
        ebc[a, a * 16:(a + 1) * 16] = 1.0
        ebc[a, 256 + a * 64:256 + (a + 1) * 64] = 1.0
        ebc[a, 1024 + a * 8:1024 + (a + 1) * 8] = 1.0

    t8 = np.zeros((8, 192), np.float32)
    for a in range(AHZ):
        for u in range(16):
            t8[u % 8, a * 16 + u] = 1.0

    tile64 = np.zeros((64, 768), np.float32)
    for a in range(AHZ):
        for i in range(64):
            tile64[i, a * 64 + i] = 1.0

    s12 = np.zeros((12, 1536), np.float32)
    for j in range(12):
        s12[j, 128 * j:128 * j + 96] = 1.0

    pswap = np.zeros((192, 192), np.float32)
    for a in range(AHZ):
        for t in range(8):
            pswap[a * 16 + 8 + t, a * 16 + t] = -1.0
            pswap[a * 16 + t, a * 16 + 8 + t] = 1.0

    rope_perm = np.zeros(192, np.int32)
    for a in range(AHZ):
        for t in range(8):
            rope_perm[a * 16 + t] = a * 16 + 2 * t
            rope_perm[a * 16 + 8 + t] = a * 16 + 2 * t + 1

    qk_perm = np.zeros(144, np.int32)
    for d in range(3):
        for a in range(AHZ):
            for p in range(QPZ):
                qk_perm[d * 48 + a * QPZ + p] = a * QPZ * 3 + p * 3 + d
    v_perm = np.zeros(288, np.int32)
    for d in range(3):
        for a in range(AHZ):
            for v in range(VPZ):
                v_perm[d * 96 + a * VPZ + v] = a * VPZ * 3 + v * 3 + d

    colperm = np.arange(1344)
    for d in range(3):
        for a in range(AHZ):
            for v in range(VPZ):
                colperm[960 + d * 96 + a * VPZ + v] = 960 + a * VPZ * 3 + v * 3 + d

    return (b192, b48, ebc, t8, tile64, s12, pswap, rope_perm, qk_perm,
            v_perm, colperm)


_CONSTS = _prep_consts()


def _weights(q1_w, q2_w, k1_w, k2_w, v1_w, v2_w, bia_w, back_w, back_b,
             gamma, ln_g, ln_b):
    (b192, b48, ebc, t8, tile64, s12, pswap, rope_perm, qk_perm,
     v_perm, colperm) = _CONSTS
    psw = jnp.asarray(pswap)

    wk1 = k1_w[rope_perm].T
    wsrc = jnp.concatenate(
        [wk1, v1_w.T, k2_w[qk_perm].T, v2_w[v_perm].T, _dot(wk1, psw)],
        axis=1)
    wq1 = q1_w[rope_perm].T
    wdst = jnp.concatenate([wq1, _dot(wq1, psw), q2_w[qk_perm].T], axis=1)
    bwp = back_w[:, colperm].T

    b192s = jnp.asarray(b192) * (WL * 0.25)
    b48s = jnp.asarray(b48) * (gamma.reshape(1, AHZ) * (-0.1 * WC * WL))
    biaTs = bia_w.T * WL

    return (wsrc, wdst, biaTs, bwp, back_b.reshape(1, IFZ),
            ln_g.reshape(1, IFZ), ln_b.reshape(1, IFZ),
            b192s, b48s, jnp.asarray(t8), jnp.asarray(tile64),
            jnp.asarray(s12), jnp.asarray(ebc))


def _tc_call(g, x1h, x2fh, affdh, pedh, wts, nblk):
    grid, in_specs, out_spec, out_shape = _tc_specs(nblk)
    return pl.pallas_call(
        _tc_body, grid=grid, in_specs=in_specs, out_specs=out_spec,
        out_shape=out_shape,
        compiler_params=pltpu.CompilerParams(
            vmem_limit_bytes=112 * 1024 * 1024),
    )(g, x1h, x2fh, affdh, pedh, *wts)


def kernel(x1, x2, affines, pos_emb, edge_index, q1_w, k1_w, v1_w, q2_w,
           k2_w, v2_w, bia_w, back_w, back_b, gamma, ln_g, ln_b):
    rot = affines[:, :3, :3].reshape(NN, 9)
    trans = affines[:, :3, 3]
    table = jnp.concatenate(
        [x1, rot, trans, pos_emb,
         jnp.zeros((NN, FP - IFZ - 20), jnp.float32)], axis=1)

    idxT = edge_index.T.astype(jnp.int32)
    idx = jnp.pad(idxT, ((0, 0), (0, NPAD - NN))).reshape(ROWS)
    wts = _weights(q1_w, q2_w, k1_w, k2_w, v1_w, v2_w, bia_w, back_w,
                   back_b, gamma, ln_g, ln_b)

    g = _sc_gather(table, idx, ROWS).reshape(KK, NPAD, FP)
    return _tc_call(g, x1, x2.reshape(NN, KK * 64), affines.reshape(NN, 16),
                    pos_emb, wts, NBLK)


# speedup vs baseline: 1.1284x; 1.0816x over previous
import functools
import math

import jax
import jax.numpy as jnp
import numpy as np
from jax import lax
from jax.experimental import pallas as pl
from jax.experimental.pallas import tpu as pltpu
from jax.experimental.pallas import tpu_sc as plsc

NN = 10000
KK = 16
IFZ = 128
AHZ = 12
AFZ = 16
QPZ = 4
VPZ = 8

FP = 256
NPAD = 10240
ROWS = KK * NPAD
NW = 32
CHUNK = 128
PER_W = ROWS // NW
NCH = PER_W // CHUNK

BN = 200
NBLK = NN // BN
BE = KK * BN

WC = math.sqrt(2.0 / (9.0 * QPZ))
WL = math.sqrt(1.0 / 3.0)


def _sc_gather(table, idx, rows):
    mesh = plsc.VectorSubcoreMesh(core_axis_name="c", subcore_axis_name="s")
    per_w = rows // NW
    nch = per_w // CHUNK

    @functools.partial(
        pl.kernel,
        mesh=mesh,
        out_type=jax.ShapeDtypeStruct((rows, FP), jnp.float32),
        scratch_types=[
            pltpu.VMEM((per_w,), jnp.int32),
            pltpu.VMEM((CHUNK, FP), jnp.float32),
            pltpu.VMEM((CHUNK, FP), jnp.float32),
            pltpu.SemaphoreType.DMA,
            pltpu.SemaphoreType.DMA,
            pltpu.SemaphoreType.DMA,
            pltpu.SemaphoreType.DMA,
        ],
    )
    def k(table_hbm, idx_hbm, out_hbm, idx_all, r0, r1, sg0, sg1, so0, so1):
        wid = lax.axis_index("s") * 2 + lax.axis_index("c")
        base = wid * per_w
        pltpu.sync_copy(idx_hbm.at[pl.ds(base, per_w)], idx_all)

        def ix(c):
            return idx_all.at[pl.ds(c * CHUNK, CHUNK)]

        def start_gather(c, r, sg):
            pltpu.async_copy(table_hbm.at[ix(c)], r, sg)

        def wait_gather(c, r, sg):
            pltpu.make_async_copy(table_hbm.at[ix(c)], r, sg).wait()

        def start_out(c, r, so):
            pltpu.async_copy(r, out_hbm.at[pl.ds(base + c * CHUNK, CHUNK)], so)

        def wait_out(c, r, so):
            pltpu.make_async_copy(
                r, out_hbm.at[pl.ds(base + c * CHUNK, CHUNK)], so).wait()

        npair = nch // 2
        start_gather(0, r0, sg0)

        def body(p, carry):
            c0 = 2 * p
            wait_gather(c0, r0, sg0)

            @pl.when(p > 0)
            def _():
                wait_out(c0 - 1, r1, so1)

            start_gather(c0 + 1, r1, sg1)
            start_out(c0, r0, so0)
            wait_gather(c0 + 1, r1, sg1)
            wait_out(c0, r0, so0)

            @pl.when(p < npair - 1)
            def _():
                start_gather(c0 + 2, r0, sg0)

            start_out(c0 + 1, r1, so1)
            return carry

        lax.fori_loop(0, npair, body, 0)
        wait_out(nch - 1, r1, so1)

    return k(table, idx)


def _dot(a, b):
    return jnp.dot(a, b, preferred_element_type=jnp.float32)


def _ksum(arr):
    acc = arr[0:BN]
    for k in range(1, KK):
        acc = acc + arr[k * BN:(k + 1) * BN]
    return acc


def _tc_body(g, x1d, x2f, affd, ped, wsrc, wdst, biaTs, bwp, bb,
             lng, lnb, b192s, b48s, t8, tile64, s12, ebc, o_ref):
    x1 = x1d[...]
    aff = affd[...]

    def A(i):
        return aff[:, i:i + 1]

    gall = g[...].reshape(BE, FP)
    xe = gall[:, 0:IFZ]
    aux = gall[:, 128:140]
    pe = gall[:, 140:148]

    proj = _dot(xe, wsrc[...])
    cT = _dot(jnp.cos(pe), t8[...])
    sT = _dot(jnp.sin(pe), t8[...])
    rk1 = proj[:, 0:192] * cT + proj[:, 816:1008] * sT

    qp = _dot(x1, wdst[...])
    cTd = _dot(jnp.cos(ped[...]), t8[...])
    sTd = _dot(jnp.sin(ped[...]), t8[...])
    rq1d = qp[:, 0:192] * cTd + qp[:, 192:384] * sTd
    rq1 = jnp.concatenate([rq1d] * KK, axis=0)
    qpt = jnp.concatenate([qp] * KK, axis=0)

    rb = _dot(aux, s12[...])

    def RB(j, w):
        return rb[:, 128 * j:128 * j + w]

    ux = qpt[:, 384:432] - proj[:, 384:432]
    uy = qpt[:, 432:480] - proj[:, 432:480]
    uz = qpt[:, 480:528] - proj[:, 480:528]

    g00 = A(0) * A(0) + A(4) * A(4) + A(8) * A(8)
    g11 = A(1) * A(1) + A(5) * A(5) + A(9) * A(9)
    g22 = A(2) * A(2) + A(6) * A(6) + A(10) * A(10)
    g01 = A(0) * A(1) + A(4) * A(5) + A(8) * A(9)
    g02 = A(0) * A(2) + A(4) * A(6) + A(8) * A(10)
    g12 = A(1) * A(2) + A(5) * A(6) + A(9) * A(10)
    gv = jnp.concatenate(
        [g00, g11, g22, 2.0 * g01, 2.0 * g02, 2.0 * g12], axis=1)
    gt = jnp.concatenate([gv] * KK, axis=0)

    x2r = x2f[...]
    x2all = jnp.concatenate(
        [x2r[:, 64 * k:64 * (k + 1)] for k in range(KK)], axis=0)

    s1 = _dot(rq1 * rk1, b192s[...])
    b48m = b48s[...]
    s2 = (gt[:, 0:1] * _dot(ux * ux, b48m)
          + gt[:, 1:2] * _dot(uy * uy, b48m)
          + gt[:, 2:3] * _dot(uz * uz, b48m)
          + gt[:, 3:4] * _dot(ux * uy, b48m)
          + gt[:, 4:5] * _dot(ux * uz, b48m)
          + gt[:, 5:6] * _dot(uy * uz, b48m))
    bias = _dot(x2all, biaTs[...])
    e = jnp.exp(s1 + s2 + bias)

    w_bc = _dot(e, ebc[...])
    p1 = w_bc[:, 0:192] * proj[:, 192:384]
    tvx = RB(0, 96) * proj[:, 528:624] + RB(1, 96) * proj[:, 624:720] \
        + RB(2, 96) * proj[:, 720:816] + RB(9, 96)
    tvy = RB(3, 96) * proj[:, 528:624] + RB(4, 96) * proj[:, 624:720] \
        + RB(5, 96) * proj[:, 720:816] + RB(10, 96)
    tvz = RB(6, 96) * proj[:, 528:624] + RB(7, 96) * proj[:, 624:720] \
        + RB(8, 96) * proj[:, 720:816] + RB(11, 96)
    pvx = w_bc[:, 1024:1120] * tvx
    pvy = w_bc[:, 1024:1120] * tvy
    pvz = w_bc[:, 1024:1120] * tvz
    p2 = w_bc[:, 256:1024] * _dot(x2all, tile64[...])

    denom = _ksum(e)
    acc1 = _ksum(p1)
    avx = _ksum(pvx)
    avy = _ksum(pvy)
    avz = _ksum(pvz)
    acc2 = _ksum(p2)

    winv = 1.0 / denom
    v_bc = _dot(winv, ebc[...])
    out1 = acc1 * v_bc[:, 0:192]
    out2 = acc2 * v_bc[:, 256:1024]
    w96 = v_bc[:, 1024:1120]
    yx = avx * w96 - A(3)
    yy = avy * w96 - A(7)
    yz = avz * w96 - A(11)
    ox = A(0) * yx + A(4) * yy + A(8) * yz
    oy = A(1) * yx + A(5) * yy + A(9) * yz
    oz = A(2) * yx + A(6) * yy + A(10) * yz
    nrm = jnp.sqrt(ox * ox + oy * oy + oz * oz + 1e-12)

    cat = jnp.concatenate([out1, out2, ox, oy, oz, nrm], axis=1)
    out = _dot(cat, bwp[...]) + bb[...]
    h = math.sqrt(2.0) * x1 + out
    mu = jnp.mean(h, axis=-1, keepdims=True)
    var = jnp.mean(jnp.square(h - mu), axis=-1, keepdims=True)
    o_ref[...] = lng[...] * (h - mu) * jax.lax.rsqrt(var + 1e-5) + lnb[...]


def _tc_specs(nblk):
    def blk(shape, imap):
        return pl.BlockSpec(shape, imap)

    def c2(shape):
        return pl.BlockSpec(shape, lambda i: (0, 0))

    in_specs = [
        blk((KK, BN, FP), lambda i: (0, i, 0)),
        blk((BN, IFZ), lambda i: (i, 0)),
        blk((BN, KK * 64), lambda i: (i, 0)),
        blk((BN, 16), lambda i: (i, 0)),
        blk((BN, 8), lambda i: (i, 0)),
        c2((IFZ, 1008)),
        c2((IFZ, 528)),
        c2((64, AHZ)),
        c2((1344, IFZ)),
        c2((1, IFZ)),
        c2((1, IFZ)),
        c2((1, IFZ)),
        c2((192, AHZ)),
        c2((48, AHZ)),
        c2((8, 192)),
        c2((64, 768)),
        c2((12, 1536)),
        c2((AHZ, 1120)),
    ]
    out_spec = pl.BlockSpec((BN, IFZ), lambda i: (i, 0))
    out_shape = jax.ShapeDtypeStruct((nblk * BN, IFZ), jnp.float32)
    return (nblk,), in_specs, out_spec, out_shape


def _prep_consts():
    b192 = (np.arange(192)[:, None] // AFZ == np.arange(AHZ)[None, :]
            ).astype(np.float32)
    b48 = (np.arange(48)[:, None] // QPZ == np.arange(AHZ)[None, :]
           ).astype(np.float32)

    ebc = np.zeros((AHZ, 1120), np.float32)
    for a in range(AHZ):
        ebc[a, a * 16:(a + 1) * 16] = 1.0
        ebc[a, 256 + a * 64:256 + (a + 1) * 64] = 1.0
        ebc[a, 1024 + a * 8:1024 + (a + 1) * 8] = 1.0

    t8 = np.zeros((8, 192), np.float32)
    for a in range(AHZ):
        for u in range(16):
            t8[u % 8, a * 16 + u] = 1.0

    tile64 = np.zeros((64, 768), np.float32)
    for a in range(AHZ):
        for i in range(64):
            tile64[i, a * 64 + i] = 1.0

    s12 = np.zeros((12, 1536), np.float32)
    for j in range(12):
        s12[j, 128 * j:128 * j + 96] = 1.0

    pswap = np.zeros((192, 192), np.float32)
    for a in range(AHZ):
        for t in range(8):
            pswap[a * 16 + 8 + t, a * 16 + t] = -1.0
            pswap[a * 16 + t, a * 16 + 8 + t] = 1.0

    rope_perm = np.zeros(192, np.int32)
    for a in range(AHZ):
        for t in range(8):
            rope_perm[a * 16 + t] = a * 16 + 2 * t
            rope_perm[a * 16 + 8 + t] = a * 16 + 2 * t + 1

    qk_perm = np.zeros(144, np.int32)
    for d in range(3):
        for a in range(AHZ):
            for p in range(QPZ):
                qk_perm[d * 48 + a * QPZ + p] = a * QPZ * 3 + p * 3 + d
    v_perm = np.zeros(288, np.int32)
    for d in range(3):
        for a in range(AHZ):
            for v in range(VPZ):
                v_perm[d * 96 + a * VPZ + v] = a * VPZ * 3 + v * 3 + d

    colperm = np.arange(1344)
    for d in range(3):
        for a in range(AHZ):
            for v in range(VPZ):
                colperm[960 + d * 96 + a * VPZ + v] = 960 + a * VPZ * 3 + v * 3 + d

    return (b192, b48, ebc, t8, tile64, s12, pswap, rope_perm, qk_perm,
            v_perm, colperm)


_CONSTS = _prep_consts()


def _weights(q1_w, q2_w, k1_w, k2_w, v1_w, v2_w, bia_w, back_w, back_b,
             gamma, ln_g, ln_b):
    (b192, b48, ebc, t8, tile64, s12, pswap, rope_perm, qk_perm,
     v_perm, colperm) = _CONSTS
    psw = jnp.asarray(pswap)

    wk1 = k1_w[rope_perm].T
    wsrc = jnp.concatenate(
        [wk1, v1_w.T, k2_w[qk_perm].T, v2_w[v_perm].T, _dot(wk1, psw)],
        axis=1)
    wq1 = q1_w[rope_perm].T
    wdst = jnp.concatenate([wq1, _dot(wq1, psw), q2_w[qk_perm].T], axis=1)
    bwp = back_w[:, colperm].T

    b192s = jnp.asarray(b192) * (WL * 0.25)
    b48s = jnp.asarray(b48) * (gamma.reshape(1, AHZ) * (-0.1 * WC * WL))
    biaTs = bia_w.T * WL

    return (wsrc, wdst, biaTs, bwp, back_b.reshape(1, IFZ),
            ln_g.reshape(1, IFZ), ln_b.reshape(1, IFZ),
            b192s, b48s, jnp.asarray(t8), jnp.asarray(tile64),
            jnp.asarray(s12), jnp.asarray(ebc))


def _tc_call(g, x1h, x2fh, affdh, pedh, wts, nblk):
    grid, in_specs, out_spec, out_shape = _tc_specs(nblk)
    return pl.pallas_call(
        _tc_body, grid=grid, in_specs=in_specs, out_specs=out_spec,
        out_shape=out_shape,
        compiler_params=pltpu.CompilerParams(
            vmem_limit_bytes=112 * 1024 * 1024),
    )(g, x1h, x2fh, affdh, pedh, *wts)


def kernel(x1, x2, affines, pos_emb, edge_index, q1_w, k1_w, v1_w, q2_w,
           k2_w, v2_w, bia_w, back_w, back_b, gamma, ln_g, ln_b):
    rot = affines[:, :3, :3].reshape(NN, 9)
    trans = affines[:, :3, 3]
    table = jnp.concatenate(
        [x1, rot, trans, pos_emb,
         jnp.zeros((NN, FP - IFZ - 20), jnp.float32)], axis=1)

    idxT = edge_index.T.astype(jnp.int32)
    idx = jnp.pad(idxT, ((0, 0), (0, NPAD - NN))).reshape(ROWS)
    wts = _weights(q1_w, q2_w, k1_w, k2_w, v1_w, v2_w, bia_w, back_w,
                   back_b, gamma, ln_g, ln_b)

    g = _sc_gather(table, idx, ROWS).reshape(KK, NPAD, FP)
    return _tc_call(g, x1, x2.reshape(NN, KK * 64), affines.reshape(NN, 16),
                    pos_emb, wts, NBLK)


# speedup vs baseline: 1.1673x; 1.0344x over previous
import functools
import math

import jax
import jax.numpy as jnp
import numpy as np
from jax import lax
from jax.experimental import pallas as pl
from jax.experimental.pallas import tpu as pltpu
from jax.experimental.pallas import tpu_sc as plsc

NN = 10000
KK = 16
IFZ = 128
AHZ = 12
AFZ = 16
QPZ = 4
VPZ = 8

FP = 256
NPAD = 10240
ROWS = KK * NPAD
NW = 32
CHUNK = 128
PER_W = ROWS // NW
NCH = PER_W // CHUNK

BN = 200
NBLK = NN // BN
BE = KK * BN

WC = math.sqrt(2.0 / (9.0 * QPZ))
WL = math.sqrt(1.0 / 3.0)


def _sc_gather(table, idx, rows):
    mesh = plsc.VectorSubcoreMesh(core_axis_name="c", subcore_axis_name="s")
    per_w = rows // NW
    nch = per_w // CHUNK

    @functools.partial(
        pl.kernel,
        mesh=mesh,
        out_type=jax.ShapeDtypeStruct((rows, 128), jnp.int32),
        scratch_types=[
            pltpu.VMEM((per_w,), jnp.int32),
            pltpu.VMEM((CHUNK, 128), jnp.int32),
            pltpu.VMEM((CHUNK, 128), jnp.int32),
            pltpu.SemaphoreType.DMA,
            pltpu.SemaphoreType.DMA,
            pltpu.SemaphoreType.DMA,
            pltpu.SemaphoreType.DMA,
        ],
    )
    def k(table_hbm, idx_hbm, out_hbm, idx_all, r0, r1, sg0, sg1, so0, so1):
        wid = lax.axis_index("s") * 2 + lax.axis_index("c")
        base = wid * per_w
        pltpu.sync_copy(idx_hbm.at[pl.ds(base, per_w)], idx_all)

        def ix(c):
            return idx_all.at[pl.ds(c * CHUNK, CHUNK)]

        def start_gather(c, r, sg):
            pltpu.async_copy(table_hbm.at[ix(c)], r, sg)

        def wait_gather(c, r, sg):
            pltpu.make_async_copy(table_hbm.at[ix(c)], r, sg).wait()

        def start_out(c, r, so):
            pltpu.async_copy(r, out_hbm.at[pl.ds(base + c * CHUNK, CHUNK)], so)

        def wait_out(c, r, so):
            pltpu.make_async_copy(
                r, out_hbm.at[pl.ds(base + c * CHUNK, CHUNK)], so).wait()

        npair = nch // 2
        start_gather(0, r0, sg0)

        def body(p, carry):
            c0 = 2 * p
            wait_gather(c0, r0, sg0)

            @pl.when(p > 0)
            def _():
                wait_out(c0 - 1, r1, so1)

            start_gather(c0 + 1, r1, sg1)
            start_out(c0, r0, so0)
            wait_gather(c0 + 1, r1, sg1)
            wait_out(c0, r0, so0)

            @pl.when(p < npair - 1)
            def _():
                start_gather(c0 + 2, r0, sg0)

            start_out(c0 + 1, r1, so1)
            return carry

        lax.fori_loop(0, npair, body, 0)
        wait_out(nch - 1, r1, so1)

    return k(table, idx)


def _dot(a, b):
    return jnp.dot(a, b, preferred_element_type=jnp.float32)


def _ksum(arr):
    acc = arr[0:BN]
    for k in range(1, KK):
        acc = acc + arr[k * BN:(k + 1) * BN]
    return acc


def _tc_body(g, x1d, x2f, affd, ped, wsrc, wdst, biaTs, bwp, bb,
             lng, lnb, b192s, b48s, t8, tile64, s12, ebc, o_ref):
    x1 = x1d[...]
    aff = affd[...]

    def A(i):
        return aff[:, i:i + 1]

    gi = g[...].reshape(BE, 128)
    xe = jax.lax.bitcast_convert_type(gi << 16, jnp.float32)
    aux_plane = jax.lax.bitcast_convert_type(
        gi & jnp.int32(-65536), jnp.float32)
    aux = aux_plane[:, 0:12]
    pe = aux_plane[:, 12:20]

    proj = _dot(xe, wsrc[...])
    cT = _dot(jnp.cos(pe), t8[...])
    sT = _dot(jnp.sin(pe), t8[...])
    rk1 = proj[:, 0:192] * cT + proj[:, 816:1008] * sT

    qp = _dot(x1, wdst[...])
    cTd = _dot(jnp.cos(ped[...]), t8[...])
    sTd = _dot(jnp.sin(ped[...]), t8[...])
    rq1d = qp[:, 0:192] * cTd + qp[:, 192:384] * sTd
    rq1 = jnp.concatenate([rq1d] * KK, axis=0)
    qpt = jnp.concatenate([qp] * KK, axis=0)

    rb = _dot(aux, s12[...])

    def RB(j, w):
        return rb[:, 128 * j:128 * j + w]

    ux = qpt[:, 384:432] - proj[:, 384:432]
    uy = qpt[:, 432:480] - proj[:, 432:480]
    uz = qpt[:, 480:528] - proj[:, 480:528]

    g00 = A(0) * A(0) + A(4) * A(4) + A(8) * A(8)
    g11 = A(1) * A(1) + A(5) * A(5) + A(9) * A(9)
    g22 = A(2) * A(2) + A(6) * A(6) + A(10) * A(10)
    g01 = A(0) * A(1) + A(4) * A(5) + A(8) * A(9)
    g02 = A(0) * A(2) + A(4) * A(6) + A(8) * A(10)
    g12 = A(1) * A(2) + A(5) * A(6) + A(9) * A(10)
    gv = jnp.concatenate(
        [g00, g11, g22, 2.0 * g01, 2.0 * g02, 2.0 * g12], axis=1)
    gt = jnp.concatenate([gv] * KK, axis=0)

    x2r = x2f[...]
    x2all = jnp.concatenate(
        [x2r[:, 64 * k:64 * (k + 1)] for k in range(KK)], axis=0)

    s1 = _dot(rq1 * rk1, b192s[...])
    b48m = b48s[...]
    s2 = (gt[:, 0:1] * _dot(ux * ux, b48m)
          + gt[:, 1:2] * _dot(uy * uy, b48m)
          + gt[:, 2:3] * _dot(uz * uz, b48m)
          + gt[:, 3:4] * _dot(ux * uy, b48m)
          + gt[:, 4:5] * _dot(ux * uz, b48m)
          + gt[:, 5:6] * _dot(uy * uz, b48m))
    bias = _dot(x2all, biaTs[...])
    e = jnp.exp(s1 + s2 + bias)

    w_bc = _dot(e, ebc[...])
    p1 = w_bc[:, 0:192] * proj[:, 192:384]
    tvx = RB(0, 96) * proj[:, 528:624] + RB(1, 96) * proj[:, 624:720] \
        + RB(2, 96) * proj[:, 720:816] + RB(9, 96)
    tvy = RB(3, 96) * proj[:, 528:624] + RB(4, 96) * proj[:, 624:720] \
        + RB(5, 96) * proj[:, 720:816] + RB(10, 96)
    tvz = RB(6, 96) * proj[:, 528:624] + RB(7, 96) * proj[:, 624:720] \
        + RB(8, 96) * proj[:, 720:816] + RB(11, 96)
    pvx = w_bc[:, 1024:1120] * tvx
    pvy = w_bc[:, 1024:1120] * tvy
    pvz = w_bc[:, 1024:1120] * tvz
    p2 = w_bc[:, 256:1024] * _dot(x2all, tile64[...])

    denom = _ksum(e)
    acc1 = _ksum(p1)
    avx = _ksum(pvx)
    avy = _ksum(pvy)
    avz = _ksum(pvz)
    acc2 = _ksum(p2)

    winv = 1.0 / denom
    v_bc = _dot(winv, ebc[...])
    out1 = acc1 * v_bc[:, 0:192]
    out2 = acc2 * v_bc[:, 256:1024]
    w96 = v_bc[:, 1024:1120]
    yx = avx * w96 - A(3)
    yy = avy * w96 - A(7)
    yz = avz * w96 - A(11)
    ox = A(0) * yx + A(4) * yy + A(8) * yz
    oy = A(1) * yx + A(5) * yy + A(9) * yz
    oz = A(2) * yx + A(6) * yy + A(10) * yz
    nrm = jnp.sqrt(ox * ox + oy * oy + oz * oz + 1e-12)

    cat = jnp.concatenate([out1, out2, ox, oy, oz, nrm], axis=1)
    out = _dot(cat, bwp[...]) + bb[...]
    h = math.sqrt(2.0) * x1 + out
    mu = jnp.mean(h, axis=-1, keepdims=True)
    var = jnp.mean(jnp.square(h - mu), axis=-1, keepdims=True)
    o_ref[...] = lng[...] * (h - mu) * jax.lax.rsqrt(var + 1e-5) + lnb[...]


def _tc_specs(nblk):
    def blk(shape, imap):
        return pl.BlockSpec(shape, imap)

    def c2(shape):
        return pl.BlockSpec(shape, lambda i: (0, 0))

    in_specs = [
        blk((KK, BN, 128), lambda i: (0, i, 0)),
        blk((BN, IFZ), lambda i: (i, 0)),
        blk((BN, KK * 64), lambda i: (i, 0)),
        blk((BN, 16), lambda i: (i, 0)),
        blk((BN, 8), lambda i: (i, 0)),
        c2((IFZ, 1008)),
        c2((IFZ, 528)),
        c2((64, AHZ)),
        c2((1344, IFZ)),
        c2((1, IFZ)),
        c2((1, IFZ)),
        c2((1, IFZ)),
        c2((192, AHZ)),
        c2((48, AHZ)),
        c2((8, 192)),
        c2((64, 768)),
        c2((12, 1536)),
        c2((AHZ, 1120)),
    ]
    out_spec = pl.BlockSpec((BN, IFZ), lambda i: (i, 0))
    out_shape = jax.ShapeDtypeStruct((nblk * BN, IFZ), jnp.float32)
    return (nblk,), in_specs, out_spec, out_shape


def _prep_consts():
    b192 = (np.arange(192)[:, None] // AFZ == np.arange(AHZ)[None, :]
            ).astype(np.float32)
    b48 = (np.arange(48)[:, None] // QPZ == np.arange(AHZ)[None, :]
           ).astype(np.float32)

    ebc = np.zeros((AHZ, 1120), np.float32)
    for a in range(AHZ):
        ebc[a, a * 16:(a + 1) * 16] = 1.0
        ebc[a, 256 + a * 64:256 + (a + 1) * 64] = 1.0
        ebc[a, 1024 + a * 8:1024 + (a + 1) * 8] = 1.0

    t8 = np.zeros((8, 192), np.float32)
    for a in range(AHZ):
        for u in range(16):
            t8[u % 8, a * 16 + u] = 1.0

    tile64 = np.zeros((64, 768), np.float32)
    for a in range(AHZ):
        for i in range(64):
            tile64[i, a * 64 + i] = 1.0

    s12 = np.zeros((12, 1536), np.float32)
    for j in range(12):
        s12[j, 128 * j:128 * j + 96] = 1.0

    pswap = np.zeros((192, 192), np.float32)
    for a in range(AHZ):
        for t in range(8):
            pswap[a * 16 + 8 + t, a * 16 + t] = -1.0
            pswap[a * 16 + t, a * 16 + 8 + t] = 1.0

    rope_perm = np.zeros(192, np.int32)
    for a in range(AHZ):
        for t in range(8):
            rope_perm[a * 16 + t] = a * 16 + 2 * t
            rope_perm[a * 16 + 8 + t] = a * 16 + 2 * t + 1

    qk_perm = np.zeros(144, np.int32)
    for d in range(3):
        for a in range(AHZ):
            for p in range(QPZ):
                qk_perm[d * 48 + a * QPZ + p] = a * QPZ * 3 + p * 3 + d
    v_perm = np.zeros(288, np.int32)
    for d in range(3):
        for a in range(AHZ):
            for v in range(VPZ):
                v_perm[d * 96 + a * VPZ + v] = a * VPZ * 3 + v * 3 + d

    colperm = np.arange(1344)
    for d in range(3):
        for a in range(AHZ):
            for v in range(VPZ):
                colperm[960 + d * 96 + a * VPZ + v] = 960 + a * VPZ * 3 + v * 3 + d

    return (b192, b48, ebc, t8, tile64, s12, pswap, rope_perm, qk_perm,
            v_perm, colperm)


_CONSTS = _prep_consts()


def _weights(q1_w, q2_w, k1_w, k2_w, v1_w, v2_w, bia_w, back_w, back_b,
             gamma, ln_g, ln_b):
    (b192, b48, ebc, t8, tile64, s12, pswap, rope_perm, qk_perm,
     v_perm, colperm) = _CONSTS
    psw = jnp.asarray(pswap)

    wk1 = k1_w[rope_perm].T
    wsrc = jnp.concatenate(
        [wk1, v1_w.T, k2_w[qk_perm].T, v2_w[v_perm].T, _dot(wk1, psw)],
        axis=1)
    wq1 = q1_w[rope_perm].T
    wdst = jnp.concatenate([wq1, _dot(wq1, psw), q2_w[qk_perm].T], axis=1)
    bwp = back_w[:, colperm].T

    b192s = jnp.asarray(b192) * (WL * 0.25)
    b48s = jnp.asarray(b48) * (gamma.reshape(1, AHZ) * (-0.1 * WC * WL))
    biaTs = bia_w.T * WL

    return (wsrc, wdst, biaTs, bwp, back_b.reshape(1, IFZ),
            ln_g.reshape(1, IFZ), ln_b.reshape(1, IFZ),
            b192s, b48s, jnp.asarray(t8), jnp.asarray(tile64),
            jnp.asarray(s12), jnp.asarray(ebc))


def _tc_call(g, x1h, x2fh, affdh, pedh, wts, nblk):
    grid, in_specs, out_spec, out_shape = _tc_specs(nblk)
    return pl.pallas_call(
        _tc_body, grid=grid, in_specs=in_specs, out_specs=out_spec,
        out_shape=out_shape,
        compiler_params=pltpu.CompilerParams(
            vmem_limit_bytes=112 * 1024 * 1024),
    )(g, x1h, x2fh, affdh, pedh, *wts)


def kernel(x1, x2, affines, pos_emb, edge_index, q1_w, k1_w, v1_w, q2_w,
           k2_w, v2_w, bia_w, back_w, back_b, gamma, ln_g, ln_b):
    rot = affines[:, :3, :3].reshape(NN, 9)
    trans = affines[:, :3, 3]
    aux = jnp.concatenate(
        [rot, trans, pos_emb, jnp.zeros((NN, 108), jnp.float32)], axis=1)

    def b16(v):
        return jax.lax.bitcast_convert_type(
            v.astype(jnp.bfloat16), jnp.uint16).astype(jnp.uint32)

    table = ((b16(aux) << 16) | b16(x1)).astype(jnp.int32)

    idxT = edge_index.T.astype(jnp.int32)
    idx = jnp.pad(idxT, ((0, 0), (0, NPAD - NN))).reshape(ROWS)
    wts = _weights(q1_w, q2_w, k1_w, k2_w, v1_w, v2_w, bia_w, back_w,
                   back_b, gamma, ln_g, ln_b)

    g = _sc_gather(table, idx, ROWS).reshape(KK, NPAD, 128)
    return _tc_call(g, x1, x2.reshape(NN, KK * 64), affines.reshape(NN, 16),
                    pos_emb, wts, NBLK)
